# Initial kernel scaffold; baseline (speedup 1.0000x reference)
import jax
import jax.numpy as jnp
from jax.experimental import pallas as pl


def kernel(text_emb, feature_2, graph_features, edge_index, W_g1, b_g1, W_g2, b_g2, W_fc1, b_fc1, W_fc2, b_fc2):
    raise NotImplementedError("write your pallas kernel here")


# speedup vs baseline: 4.5997x; 4.5997x over previous
import jax
import jax.numpy as jnp
from jax import lax
from jax.experimental import pallas as pl
from jax.experimental.pallas import tpu as pltpu
from jax.experimental.pallas import tpu_sc as plsc

N = 10000
E = 320000
D1 = 128
D2 = 64
DH = (D1 + D2) // 2
DDEG = 16
NC = 2
NS = 16
CH = 80
EPT = E // NS
NCHUNK = EPT // CH
DEGSPLIT = NCHUNK // 2
ROWCHUNKS = N // CH
RR = (ROWCHUNKS + NS - 1) // NS


def _sc_body(xlo_hbm, xhi_hbm, src_hbm, dst_hbm, acc_hbm, deg_hbm,
             acc_sp, deg_sp, src_v, dst_v, rows_v, ones_v, degbuf_v, sem):
    c = lax.axis_index("c")
    s = lax.axis_index("s")

    zvec = jnp.zeros((16,), jnp.float32)
    ovec = jnp.ones((16,), jnp.float32)

    def fill(i, carry):
        for k in range(DH // 16):
            rows_v[i, pl.ds(16 * k, 16)] = zvec
        ones_v[i, :] = ovec
        degbuf_v[i, :] = zvec
        return carry

    lax.fori_loop(0, CH, fill, 0)

    def zinit(t, carry):
        chk = t * NS + s

        @pl.when(chk < ROWCHUNKS)
        def _():
            r0 = pl.multiple_of(chk * CH, 8)
            pltpu.sync_copy(rows_v, acc_sp.at[pl.ds(r0, CH)])
            pltpu.sync_copy(degbuf_v, deg_sp.at[pl.ds(r0, CH)])

        return carry

    lax.fori_loop(0, RR, zinit, 0)
    plsc.subcore_barrier()

    base_w = s * EPT

    def step(j, carry):
        e0 = pl.multiple_of(base_w + j * CH, 8)
        pltpu.sync_copy(src_hbm.at[pl.ds(e0, CH)], src_v)
        pltpu.sync_copy(dst_hbm.at[pl.ds(e0, CH)], dst_v)

        @pl.when(c == 0)
        def _():
            pltpu.async_copy(xlo_hbm.at[src_v], rows_v, sem).wait()

        @pl.when(c == 1)
        def _():
            pltpu.async_copy(xhi_hbm.at[src_v], rows_v, sem).wait()

        pltpu.sync_copy(rows_v, acc_sp.at[dst_v], add=True)

        @pl.when((j < DEGSPLIT) == (c == 0))
        def _():
            pltpu.sync_copy(ones_v, deg_sp.at[dst_v], add=True)

        return carry

    lax.fori_loop(0, NCHUNK, step, 0)
    plsc.subcore_barrier()

    def drain(t, carry):
        chk = t * NS + s

        @pl.when(chk < ROWCHUNKS)
        def _():
            r0 = pl.multiple_of(chk * CH, 8)
            o0 = pl.multiple_of(c * N + chk * CH, 8)
            pltpu.sync_copy(acc_sp.at[pl.ds(r0, CH)], rows_v)
            pltpu.sync_copy(rows_v, acc_hbm.at[pl.ds(o0, CH)])
            pltpu.sync_copy(deg_sp.at[pl.ds(r0, CH)], degbuf_v)
            pltpu.sync_copy(degbuf_v, deg_hbm.at[pl.ds(o0, CH)])

        return carry

    lax.fori_loop(0, RR, drain, 0)


def _sc_segment_sum(x_lo, x_hi, src, dst):
    mesh = plsc.VectorSubcoreMesh(
        core_axis_name="c", subcore_axis_name="s", num_cores=NC, num_subcores=NS)
    f = pl.kernel(
        _sc_body,
        out_type=(jax.ShapeDtypeStruct((NC * N, DH), jnp.float32),
                  jax.ShapeDtypeStruct((NC * N, DDEG), jnp.float32)),
        mesh=mesh,
        scratch_types=[
            pltpu.VMEM_SHARED((N, DH), jnp.float32),
            pltpu.VMEM_SHARED((N, DDEG), jnp.float32),
            pltpu.VMEM((CH,), jnp.int32),
            pltpu.VMEM((CH,), jnp.int32),
            pltpu.VMEM((CH, DH), jnp.float32),
            pltpu.VMEM((CH, DDEG), jnp.float32),
            pltpu.VMEM((CH, DDEG), jnp.float32),
            pltpu.SemaphoreType.DMA,
        ],
        compiler_params=pltpu.CompilerParams(use_tc_tiling_on_sc=False),
    )
    return f(x_lo, x_hi, src, dst)


BLK = 1000


def _tc_body(text_ref, acclo_ref, acchi_ref, dega_ref, degb_ref,
             wg1_ref, bg1_ref, wg2_ref, bg2_ref, wf1_ref, bf1_ref,
             wf2_ref, bf2_ref, out_ref):
    lo = acclo_ref[...]
    hi = acchi_ref[...]
    deg = dega_ref[:, 0:1] + degb_ref[:, 0:1]
    d = jnp.maximum(deg, 1.0)
    g1 = jnp.concatenate([lo, hi[:, :D1 - DH]], axis=1) / d
    g2 = hi[:, D1 - DH:] / d
    h1 = jnp.maximum(
        jnp.dot(g1, wg1_ref[...], preferred_element_type=jnp.float32)
        + bg1_ref[...], 0.0)
    h2 = jnp.maximum(
        jnp.dot(g2, wg2_ref[...], preferred_element_type=jnp.float32)
        + bg2_ref[...], 0.0)
    t = (jnp.dot(text_ref[...], wf1_ref[...], preferred_element_type=jnp.float32)
         + bf1_ref[...])
    w2 = wf2_ref[...]
    out = jnp.dot(t, w2[:D1], preferred_element_type=jnp.float32)
    out = out + jnp.dot(h1, w2[D1:2 * D1], preferred_element_type=jnp.float32)
    out = out + jnp.dot(h2, w2[2 * D1:], preferred_element_type=jnp.float32)
    out_ref[...] = out + bf2_ref[...]


def _tc_combine(text_emb, acc_lo, acc_hi, deg_a, deg_b,
                W_g1, b_g1, W_g2, b_g2, W_fc1, b_fc1, W_fc2, b_fc2):
    row = lambda i: (i, 0)
    full = lambda i: (0, 0)
    return pl.pallas_call(
        _tc_body,
        grid=(N // BLK,),
        in_specs=[
            pl.BlockSpec((BLK, 256), row),
            pl.BlockSpec((BLK, DH), row),
            pl.BlockSpec((BLK, DH), row),
            pl.BlockSpec((BLK, DDEG), row),
            pl.BlockSpec((BLK, DDEG), row),
            pl.BlockSpec((D1, D1), full),
            pl.BlockSpec((1, D1), full),
            pl.BlockSpec((D2, D2), full),
            pl.BlockSpec((1, D2), full),
            pl.BlockSpec((256, D1), full),
            pl.BlockSpec((1, D1), full),
            pl.BlockSpec((2 * D1 + D2, 256), full),
            pl.BlockSpec((1, 256), full),
        ],
        out_specs=pl.BlockSpec((BLK, 256), row),
        out_shape=jax.ShapeDtypeStruct((N, 256), jnp.float32),
    )(text_emb, acc_lo, acc_hi, deg_a, deg_b,
      W_g1, b_g1, W_g2, b_g2, W_fc1, b_fc1, W_fc2, b_fc2)


def kernel(text_emb, feature_2, graph_features, edge_index,
           W_g1, b_g1, W_g2, b_g2, W_fc1, b_fc1, W_fc2, b_fc2):
    x = jnp.concatenate([graph_features, feature_2], axis=1)
    x_lo = x[:, :DH]
    x_hi = x[:, DH:]
    src = edge_index[0]
    dst = edge_index[1]
    acc, deg = _sc_segment_sum(x_lo, x_hi, src, dst)
    return _tc_combine(
        text_emb, acc[:N], acc[N:], deg[:N], deg[N:],
        W_g1, b_g1.reshape(1, -1), W_g2, b_g2.reshape(1, -1),
        W_fc1, b_fc1.reshape(1, -1), W_fc2, b_fc2.reshape(1, -1))


# speedup vs baseline: 10.7859x; 2.3449x over previous
import jax
import jax.numpy as jnp
from jax import lax
from jax.experimental import pallas as pl
from jax.experimental.pallas import tpu as pltpu
from jax.experimental.pallas import tpu_sc as plsc

N = 10000
E = 320000
D1 = 128
D2 = 64
DH = (D1 + D2) // 2
DDEG = 16
NC = 2
NS = 16
CH = 80
EPT = E // NS
NCHUNK = EPT // CH
DEGSPLIT = NCHUNK // 2
ROWCHUNKS = N // CH
RR = (ROWCHUNKS + NS - 1) // NS
NBUF = 2


def _sc_body(xlo_hbm, xhi_hbm, src_hbm, dst_hbm, acc_hbm, deg_hbm,
             acc_sp, deg_sp, srcall_v, dstall_v, rows0_v, rows1_v,
             ones_v, degbuf_v, sem):
    c = lax.axis_index("c")
    s = lax.axis_index("s")
    rows_bufs = (rows0_v, rows1_v)

    zvec = jnp.zeros((16,), jnp.float32)
    ovec = jnp.ones((16,), jnp.float32)

    def fill(i, carry):
        for k in range(DH // 16):
            rows0_v[i, pl.ds(16 * k, 16)] = zvec
        ones_v[i, :] = ovec
        degbuf_v[i, :] = zvec
        return carry

    lax.fori_loop(0, CH, fill, 0)

    r0 = pl.multiple_of(s * NCHUNK, 2)
    pltpu.sync_copy(src_hbm.at[pl.ds(r0, NCHUNK)], srcall_v)
    pltpu.sync_copy(dst_hbm.at[pl.ds(r0, NCHUNK)], dstall_v)

    def zinit(t, carry):
        chk = t * NS + s

        @pl.when(chk < ROWCHUNKS)
        def _():
            a0 = pl.multiple_of(chk * CH, 8)
            pltpu.sync_copy(rows0_v, acc_sp.at[pl.ds(a0, CH)])
            pltpu.sync_copy(degbuf_v, deg_sp.at[pl.ds(a0, CH)])

        return carry

    lax.fori_loop(0, RR, zinit, 0)
    plsc.subcore_barrier()

    def start_gather(j, buf):
        @pl.when(c == 0)
        def _():
            pltpu.async_copy(xlo_hbm.at[srcall_v.at[j]], buf, sem)

        @pl.when(c == 1)
        def _():
            pltpu.async_copy(xhi_hbm.at[srcall_v.at[j]], buf, sem)

    def wait_gather(buf):
        pltpu.make_async_copy(xlo_hbm.at[srcall_v.at[0]], buf, sem).wait()

    start_gather(0, rows_bufs[0])

    def outer(g, carry):
        for b in range(NBUF):
            j = g * NBUF + b

            @pl.when(j + 1 < NCHUNK)
            def _():
                start_gather(j + 1, rows_bufs[(b + 1) % NBUF])

            wait_gather(rows_bufs[b])
            pltpu.sync_copy(rows_bufs[b], acc_sp.at[dstall_v.at[j]], add=True)

            @pl.when((j < DEGSPLIT) == (c == 0))
            def _():
                pltpu.sync_copy(ones_v, deg_sp.at[dstall_v.at[j]], add=True)

        return carry

    lax.fori_loop(0, NCHUNK // NBUF, outer, 0)
    plsc.subcore_barrier()

    def drain(t, carry):
        chk = t * NS + s

        @pl.when(chk < ROWCHUNKS)
        def _():
            a0 = pl.multiple_of(chk * CH, 8)
            o0 = pl.multiple_of(c * N + chk * CH, 8)
            pltpu.sync_copy(acc_sp.at[pl.ds(a0, CH)], rows0_v)
            pltpu.sync_copy(rows0_v, acc_hbm.at[pl.ds(o0, CH)])
            pltpu.sync_copy(deg_sp.at[pl.ds(a0, CH)], degbuf_v)
            pltpu.sync_copy(degbuf_v, deg_hbm.at[pl.ds(o0, CH)])

        return carry

    lax.fori_loop(0, RR, drain, 0)


def _sc_segment_sum(x_lo, x_hi, src2d, dst2d):
    mesh = plsc.VectorSubcoreMesh(
        core_axis_name="c", subcore_axis_name="s", num_cores=NC, num_subcores=NS)
    f = pl.kernel(
        _sc_body,
        out_type=(jax.ShapeDtypeStruct((NC * N, DH), jnp.float32),
                  jax.ShapeDtypeStruct((NC * N, DDEG), jnp.float32)),
        mesh=mesh,
        scratch_types=[
            pltpu.VMEM_SHARED((N, DH), jnp.float32),
            pltpu.VMEM_SHARED((N, DDEG), jnp.float32),
            pltpu.VMEM((NCHUNK, CH), jnp.int32),
            pltpu.VMEM((NCHUNK, CH), jnp.int32),
            pltpu.VMEM((CH, DH), jnp.float32),
            pltpu.VMEM((CH, DH), jnp.float32),
            pltpu.VMEM((CH, DDEG), jnp.float32),
            pltpu.VMEM((CH, DDEG), jnp.float32),
            pltpu.SemaphoreType.DMA,
        ],
        compiler_params=pltpu.CompilerParams(use_tc_tiling_on_sc=False),
    )
    return f(x_lo, x_hi, src2d, dst2d)


BLK = 1000


def _tc_body(text_ref, acclo_ref, acchi_ref, dega_ref, degb_ref,
             wg1_ref, bg1_ref, wg2_ref, bg2_ref, wf1_ref, bf1_ref,
             wf2_ref, bf2_ref, out_ref):
    lo = acclo_ref[...]
    hi = acchi_ref[...]
    deg = dega_ref[:, 0:1] + degb_ref[:, 0:1]
    d = jnp.maximum(deg, 1.0)
    g1 = jnp.concatenate([lo, hi[:, :D1 - DH]], axis=1) / d
    g2 = hi[:, D1 - DH:] / d
    h1 = jnp.maximum(
        jnp.dot(g1, wg1_ref[...], preferred_element_type=jnp.float32)
        + bg1_ref[...], 0.0)
    h2 = jnp.maximum(
        jnp.dot(g2, wg2_ref[...], preferred_element_type=jnp.float32)
        + bg2_ref[...], 0.0)
    t = (jnp.dot(text_ref[...], wf1_ref[...], preferred_element_type=jnp.float32)
         + bf1_ref[...])
    w2 = wf2_ref[...]
    out = jnp.dot(t, w2[:D1], preferred_element_type=jnp.float32)
    out = out + jnp.dot(h1, w2[D1:2 * D1], preferred_element_type=jnp.float32)
    out = out + jnp.dot(h2, w2[2 * D1:], preferred_element_type=jnp.float32)
    out_ref[...] = out + bf2_ref[...]


def _tc_combine(text_emb, acc, deg,
                W_g1, b_g1, W_g2, b_g2, W_fc1, b_fc1, W_fc2, b_fc2):
    nb = N // BLK
    row = lambda i: (i, 0)
    row_hi = lambda i: (nb + i, 0)
    full = lambda i: (0, 0)
    return pl.pallas_call(
        _tc_body,
        grid=(nb,),
        in_specs=[
            pl.BlockSpec((BLK, 256), row),
            pl.BlockSpec((BLK, DH), row),
            pl.BlockSpec((BLK, DH), row_hi),
            pl.BlockSpec((BLK, DDEG), row),
            pl.BlockSpec((BLK, DDEG), row_hi),
            pl.BlockSpec((D1, D1), full),
            pl.BlockSpec((1, D1), full),
            pl.BlockSpec((D2, D2), full),
            pl.BlockSpec((1, D2), full),
            pl.BlockSpec((256, D1), full),
            pl.BlockSpec((1, D1), full),
            pl.BlockSpec((2 * D1 + D2, 256), full),
            pl.BlockSpec((1, 256), full),
        ],
        out_specs=pl.BlockSpec((BLK, 256), row),
        out_shape=jax.ShapeDtypeStruct((N, 256), jnp.float32),
    )(text_emb, acc, acc, deg, deg,
      W_g1, b_g1, W_g2, b_g2, W_fc1, b_fc1, W_fc2, b_fc2)


def kernel(text_emb, feature_2, graph_features, edge_index,
           W_g1, b_g1, W_g2, b_g2, W_fc1, b_fc1, W_fc2, b_fc2):
    x = jnp.concatenate([graph_features, feature_2], axis=1)
    x_lo = x[:, :DH]
    x_hi = x[:, DH:]
    src2d = edge_index[0].reshape(E // CH, CH)
    dst2d = edge_index[1].reshape(E // CH, CH)
    acc, deg = _sc_segment_sum(x_lo, x_hi, src2d, dst2d)
    return _tc_combine(
        text_emb, acc, deg,
        W_g1, b_g1.reshape(1, -1), W_g2, b_g2.reshape(1, -1),
        W_fc1, b_fc1.reshape(1, -1), W_fc2, b_fc2.reshape(1, -1))


# speedup vs baseline: 10.9166x; 1.0121x over previous
import jax
import jax.numpy as jnp
from jax import lax
from jax.experimental import pallas as pl
from jax.experimental.pallas import tpu as pltpu
from jax.experimental.pallas import tpu_sc as plsc

N = 10000
E = 320000
D1 = 128
D2 = 64
DH = (D1 + D2) // 2
DDEG = 16
NC = 2
NS = 16
CH = 80
EPT = E // NS
NCHUNK = EPT // CH
DEGSPLIT = NCHUNK // 2
ROWCHUNKS = N // CH
RR = (ROWCHUNKS + NS - 1) // NS
NBUF = 2


def _sc_body(xlo_hbm, xhi_hbm, src_hbm, dst_hbm, acc_hbm, deg_hbm,
             acc_sp, deg_sp, srcall_v, dstall_v, rows0_v, rows1_v,
             ones_v, degbuf_v, sem_g, sem_s, sem_d):
    c = lax.axis_index("c")
    s = lax.axis_index("s")
    rows_bufs = (rows0_v, rows1_v)

    zvec = jnp.zeros((16,), jnp.float32)
    ovec = jnp.ones((16,), jnp.float32)

    def fill(i, carry):
        for k in range(DH // 16):
            rows0_v[i, pl.ds(16 * k, 16)] = zvec
        ones_v[i, :] = ovec
        degbuf_v[i, :] = zvec
        return carry

    lax.fori_loop(0, CH, fill, 0)

    r0 = pl.multiple_of(s * NCHUNK, 2)
    pltpu.sync_copy(src_hbm.at[pl.ds(r0, NCHUNK)], srcall_v)
    pltpu.sync_copy(dst_hbm.at[pl.ds(r0, NCHUNK)], dstall_v)

    def zinit(t, carry):
        chk = t * NS + s

        @pl.when(chk < ROWCHUNKS)
        def _():
            a0 = pl.multiple_of(chk * CH, 8)
            pltpu.sync_copy(rows0_v, acc_sp.at[pl.ds(a0, CH)])
            pltpu.sync_copy(degbuf_v, deg_sp.at[pl.ds(a0, CH)])

        return carry

    lax.fori_loop(0, RR, zinit, 0)
    plsc.subcore_barrier()

    def start_gather(j, buf):
        @pl.when(c == 0)
        def _():
            pltpu.async_copy(xlo_hbm.at[srcall_v.at[j]], buf, sem_g)

        @pl.when(c == 1)
        def _():
            pltpu.async_copy(xhi_hbm.at[srcall_v.at[j]], buf, sem_g)

    def wait_gather(buf):
        pltpu.make_async_copy(xlo_hbm.at[srcall_v.at[0]], buf, sem_g).wait()

    def start_scatter(j, buf):
        pltpu.async_copy(buf, acc_sp.at[dstall_v.at[j]], sem_s, add=True)

    def wait_scatter(j, buf):
        pltpu.make_async_copy(buf, acc_sp.at[dstall_v.at[j]], sem_s).wait()

    def start_deg(j):
        pltpu.async_copy(ones_v, deg_sp.at[dstall_v.at[j]], sem_d, add=True)

    def wait_deg(j):
        pltpu.make_async_copy(ones_v, deg_sp.at[dstall_v.at[j]], sem_d).wait()

    def own_deg(j):
        return (j < DEGSPLIT) == (c == 0)

    start_gather(0, rows_bufs[0])

    def outer(g, carry):
        for b in range(NBUF):
            j = g * NBUF + b

            @pl.when(j >= 1)
            def _():
                wait_scatter(j - 1, rows_bufs[(b + 1) % NBUF])

            @pl.when(j + 1 < NCHUNK)
            def _():
                start_gather(j + 1, rows_bufs[(b + 1) % NBUF])

            wait_gather(rows_bufs[b])
            start_scatter(j, rows_bufs[b])

            @pl.when(own_deg(j))
            def _():
                start_deg(j)

            @pl.when(own_deg(j - 2) & (j >= 2))
            def _():
                wait_deg(j - 2)

        return carry

    lax.fori_loop(0, NCHUNK // NBUF, outer, 0)
    wait_scatter(NCHUNK - 1, rows_bufs[(NCHUNK - 1) % NBUF])

    @pl.when(c == 1)
    def _():
        wait_deg(NCHUNK - 2)
        wait_deg(NCHUNK - 1)

    plsc.subcore_barrier()

    def drain(t, carry):
        chk = t * NS + s

        @pl.when(chk < ROWCHUNKS)
        def _():
            a0 = pl.multiple_of(chk * CH, 8)
            o0 = pl.multiple_of(c * N + chk * CH, 8)
            pltpu.sync_copy(acc_sp.at[pl.ds(a0, CH)], rows0_v)
            pltpu.sync_copy(rows0_v, acc_hbm.at[pl.ds(o0, CH)])
            pltpu.sync_copy(deg_sp.at[pl.ds(a0, CH)], degbuf_v)
            pltpu.sync_copy(degbuf_v, deg_hbm.at[pl.ds(o0, CH)])

        return carry

    lax.fori_loop(0, RR, drain, 0)


def _sc_segment_sum(x_lo, x_hi, src2d, dst2d):
    mesh = plsc.VectorSubcoreMesh(
        core_axis_name="c", subcore_axis_name="s", num_cores=NC, num_subcores=NS)
    f = pl.kernel(
        _sc_body,
        out_type=(jax.ShapeDtypeStruct((NC * N, DH), jnp.float32),
                  jax.ShapeDtypeStruct((NC * N, DDEG), jnp.float32)),
        mesh=mesh,
        scratch_types=[
            pltpu.VMEM_SHARED((N, DH), jnp.float32),
            pltpu.VMEM_SHARED((N, DDEG), jnp.float32),
            pltpu.VMEM((NCHUNK, CH), jnp.int32),
            pltpu.VMEM((NCHUNK, CH), jnp.int32),
            pltpu.VMEM((CH, DH), jnp.float32),
            pltpu.VMEM((CH, DH), jnp.float32),
            pltpu.VMEM((CH, DDEG), jnp.float32),
            pltpu.VMEM((CH, DDEG), jnp.float32),
            pltpu.SemaphoreType.DMA,
            pltpu.SemaphoreType.DMA,
            pltpu.SemaphoreType.DMA,
        ],
        compiler_params=pltpu.CompilerParams(use_tc_tiling_on_sc=False),
    )
    return f(x_lo, x_hi, src2d, dst2d)


BLK = 1000


def _tc_body(text_ref, acclo_ref, acchi_ref, dega_ref, degb_ref,
             wg1_ref, bg1_ref, wg2_ref, bg2_ref, wf1_ref, bf1_ref,
             wf2_ref, bf2_ref, out_ref):
    lo = acclo_ref[...]
    hi = acchi_ref[...]
    deg = dega_ref[:, 0:1] + degb_ref[:, 0:1]
    d = jnp.maximum(deg, 1.0)
    g1 = jnp.concatenate([lo, hi[:, :D1 - DH]], axis=1) / d
    g2 = hi[:, D1 - DH:] / d
    h1 = jnp.maximum(
        jnp.dot(g1, wg1_ref[...], preferred_element_type=jnp.float32)
        + bg1_ref[...], 0.0)
    h2 = jnp.maximum(
        jnp.dot(g2, wg2_ref[...], preferred_element_type=jnp.float32)
        + bg2_ref[...], 0.0)
    t = (jnp.dot(text_ref[...], wf1_ref[...], preferred_element_type=jnp.float32)
         + bf1_ref[...])
    w2 = wf2_ref[...]
    out = jnp.dot(t, w2[:D1], preferred_element_type=jnp.float32)
    out = out + jnp.dot(h1, w2[D1:2 * D1], preferred_element_type=jnp.float32)
    out = out + jnp.dot(h2, w2[2 * D1:], preferred_element_type=jnp.float32)
    out_ref[...] = out + bf2_ref[...]


def _tc_combine(text_emb, acc, deg,
                W_g1, b_g1, W_g2, b_g2, W_fc1, b_fc1, W_fc2, b_fc2):
    nb = N // BLK
    row = lambda i: (i, 0)
    row_hi = lambda i: (nb + i, 0)
    full = lambda i: (0, 0)
    return pl.pallas_call(
        _tc_body,
        grid=(nb,),
        in_specs=[
            pl.BlockSpec((BLK, 256), row),
            pl.BlockSpec((BLK, DH), row),
            pl.BlockSpec((BLK, DH), row_hi),
            pl.BlockSpec((BLK, DDEG), row),
            pl.BlockSpec((BLK, DDEG), row_hi),
            pl.BlockSpec((D1, D1), full),
            pl.BlockSpec((1, D1), full),
            pl.BlockSpec((D2, D2), full),
            pl.BlockSpec((1, D2), full),
            pl.BlockSpec((256, D1), full),
            pl.BlockSpec((1, D1), full),
            pl.BlockSpec((2 * D1 + D2, 256), full),
            pl.BlockSpec((1, 256), full),
        ],
        out_specs=pl.BlockSpec((BLK, 256), row),
        out_shape=jax.ShapeDtypeStruct((N, 256), jnp.float32),
    )(text_emb, acc, acc, deg, deg,
      W_g1, b_g1, W_g2, b_g2, W_fc1, b_fc1, W_fc2, b_fc2)


def kernel(text_emb, feature_2, graph_features, edge_index,
           W_g1, b_g1, W_g2, b_g2, W_fc1, b_fc1, W_fc2, b_fc2):
    x = jnp.concatenate([graph_features, feature_2], axis=1)
    x_lo = x[:, :DH]
    x_hi = x[:, DH:]
    src2d = edge_index[0].reshape(E // CH, CH)
    dst2d = edge_index[1].reshape(E // CH, CH)
    acc, deg = _sc_segment_sum(x_lo, x_hi, src2d, dst2d)
    return _tc_combine(
        text_emb, acc, deg,
        W_g1, b_g1.reshape(1, -1), W_g2, b_g2.reshape(1, -1),
        W_fc1, b_fc1.reshape(1, -1), W_fc2, b_fc2.reshape(1, -1))


# speedup vs baseline: 12.5733x; 1.1518x over previous
import jax
import jax.numpy as jnp
from jax import lax
from jax.experimental import pallas as pl
from jax.experimental.pallas import tpu as pltpu
from jax.experimental.pallas import tpu_sc as plsc

N = 10000
E = 320000
D1 = 128
D2 = 64
DH = (D1 + D2) // 2
DW = DH + 16
NC = 2
NS = 16
CH = 80
EPT = E // NS
NCHUNK = EPT // CH
SEG = 50
ROWCHUNKS = N // CH
RR = (ROWCHUNKS + NS - 1) // NS
NBUF = 5


def _sc_body(xlo_hbm, xhi_hbm, src_hbm, dst_hbm, acc_hbm,
             acc_sp, srcseg_v, dstseg_v, r0_v, r1_v, r2_v, r3_v, r4_v,
             sem_g, sem_s):
    c = lax.axis_index("c")
    s = lax.axis_index("s")
    rows_bufs = (r0_v, r1_v, r2_v, r3_v, r4_v)

    zvec = jnp.zeros((16,), jnp.float32)

    def fill(i, carry):
        for k in range(DW // 16):
            r0_v[i, pl.ds(16 * k, 16)] = zvec
        return carry

    lax.fori_loop(0, CH, fill, 0)

    base_row = s * NCHUNK
    pltpu.sync_copy(src_hbm.at[pl.ds(base_row, SEG)], srcseg_v)
    pltpu.sync_copy(dst_hbm.at[pl.ds(base_row, SEG)], dstseg_v)

    def zinit(t, carry):
        chk = t * NS + s

        @pl.when(chk < ROWCHUNKS)
        def _():
            a0 = pl.multiple_of(chk * CH, 8)
            pltpu.sync_copy(r0_v, acc_sp.at[pl.ds(a0, CH)])

        return carry

    lax.fori_loop(0, RR, zinit, 0)
    plsc.subcore_barrier()

    def start_gather(j, buf):
        @pl.when(c == 0)
        def _():
            pltpu.async_copy(xlo_hbm.at[srcseg_v.at[j % SEG]], buf, sem_g)

        @pl.when(c == 1)
        def _():
            pltpu.async_copy(xhi_hbm.at[srcseg_v.at[j % SEG]], buf, sem_g)

    def wait_gather(buf):
        pltpu.make_async_copy(xlo_hbm.at[srcseg_v.at[0]], buf, sem_g).wait()

    def start_scatter(j, buf):
        pltpu.async_copy(buf, acc_sp.at[dstseg_v.at[j % SEG]], sem_s, add=True)

    def wait_scatter(j, buf):
        pltpu.make_async_copy(buf, acc_sp.at[dstseg_v.at[j % SEG]], sem_s).wait()

    start_gather(0, rows_bufs[0])
    start_gather(1, rows_bufs[1])

    def outer(g, carry):
        for b in range(NBUF):
            j = g * NBUF + b

            if b == 0:
                reload_d = ((j % SEG) == 0) & (j > 0)

                @pl.when(reload_d)
                def _():
                    wait_scatter(j - 3, rows_bufs[(b - 3) % NBUF])
                    wait_scatter(j - 2, rows_bufs[(b - 2) % NBUF])
                    wait_scatter(j - 1, rows_bufs[(b - 1) % NBUF])
                    pltpu.sync_copy(dst_hbm.at[pl.ds(base_row + j, SEG)],
                                    dstseg_v)

                @pl.when(jnp.logical_not(reload_d) & (j >= 3))
                def _():
                    wait_scatter(j - 3, rows_bufs[(b - 3) % NBUF])
            elif b in (1, 2):
                @pl.when((j >= 3) & ((j % SEG) != b))
                def _():
                    wait_scatter(j - 3, rows_bufs[(b - 3) % NBUF])
            else:
                @pl.when(j >= 3)
                def _():
                    wait_scatter(j - 3, rows_bufs[(b - 3) % NBUF])

            if b == 3:
                reload_s = ((j % SEG) == SEG - 2) & (j + 2 < NCHUNK)

                @pl.when(reload_s)
                def _():
                    wait_gather(rows_bufs[b])
                    wait_gather(rows_bufs[(b + 1) % NBUF])
                    pltpu.sync_copy(src_hbm.at[pl.ds(base_row + j + 2, SEG)],
                                    srcseg_v)

            @pl.when(j + 2 < NCHUNK)
            def _():
                start_gather(j + 2, rows_bufs[(b + 2) % NBUF])

            if b == 3:
                @pl.when(jnp.logical_not(((j % SEG) == SEG - 2)
                                         & (j + 2 < NCHUNK)))
                def _():
                    wait_gather(rows_bufs[b])
            elif b == 4:
                @pl.when(jnp.logical_not(((j % SEG) == SEG - 1)
                                         & (j + 1 < NCHUNK)))
                def _():
                    wait_gather(rows_bufs[b])
            else:
                wait_gather(rows_bufs[b])

            start_scatter(j, rows_bufs[b])

        return carry

    lax.fori_loop(0, NCHUNK // NBUF, outer, 0)
    wait_scatter(NCHUNK - 3, rows_bufs[(NCHUNK - 3) % NBUF])
    wait_scatter(NCHUNK - 2, rows_bufs[(NCHUNK - 2) % NBUF])
    wait_scatter(NCHUNK - 1, rows_bufs[(NCHUNK - 1) % NBUF])
    plsc.subcore_barrier()

    def drain(t, carry):
        chk = t * NS + s

        @pl.when(chk < ROWCHUNKS)
        def _():
            a0 = pl.multiple_of(chk * CH, 8)
            o0 = pl.multiple_of(c * N + chk * CH, 8)
            pltpu.sync_copy(acc_sp.at[pl.ds(a0, CH)], r0_v)
            pltpu.sync_copy(r0_v, acc_hbm.at[pl.ds(o0, CH)])

        return carry

    lax.fori_loop(0, RR, drain, 0)


def _sc_segment_sum(x_lo, x_hi, src2d, dst2d):
    mesh = plsc.VectorSubcoreMesh(
        core_axis_name="c", subcore_axis_name="s", num_cores=NC, num_subcores=NS)
    f = pl.kernel(
        _sc_body,
        out_type=jax.ShapeDtypeStruct((NC * N, DW), jnp.float32),
        mesh=mesh,
        scratch_types=[
            pltpu.VMEM_SHARED((N, DW), jnp.float32),
            pltpu.VMEM((SEG, CH), jnp.int32),
            pltpu.VMEM((SEG, CH), jnp.int32),
            pltpu.VMEM((CH, DW), jnp.float32),
            pltpu.VMEM((CH, DW), jnp.float32),
            pltpu.VMEM((CH, DW), jnp.float32),
            pltpu.VMEM((CH, DW), jnp.float32),
            pltpu.VMEM((CH, DW), jnp.float32),
            pltpu.SemaphoreType.DMA,
            pltpu.SemaphoreType.DMA,
        ],
        compiler_params=pltpu.CompilerParams(use_tc_tiling_on_sc=False),
    )
    return f(x_lo, x_hi, src2d, dst2d)


BLK = 1000


def _tc_body(text_ref, acclo_ref, acchi_ref,
             wg1_ref, bg1_ref, wg2_ref, bg2_ref, wf1_ref, bf1_ref,
             wf2_ref, bf2_ref, out_ref):
    lo = acclo_ref[...]
    hi = acchi_ref[...]
    d = jnp.maximum(lo[:, DH:DH + 1], 1.0)
    g1 = jnp.concatenate([lo[:, :DH], hi[:, :D1 - DH]], axis=1) / d
    g2 = hi[:, D1 - DH:DH] / d
    h1 = jnp.maximum(
        jnp.dot(g1, wg1_ref[...], preferred_element_type=jnp.float32)
        + bg1_ref[...], 0.0)
    h2 = jnp.maximum(
        jnp.dot(g2, wg2_ref[...], preferred_element_type=jnp.float32)
        + bg2_ref[...], 0.0)
    t = (jnp.dot(text_ref[...], wf1_ref[...], preferred_element_type=jnp.float32)
         + bf1_ref[...])
    w2 = wf2_ref[...]
    out = jnp.dot(t, w2[:D1], preferred_element_type=jnp.float32)
    out = out + jnp.dot(h1, w2[D1:2 * D1], preferred_element_type=jnp.float32)
    out = out + jnp.dot(h2, w2[2 * D1:], preferred_element_type=jnp.float32)
    out_ref[...] = out + bf2_ref[...]


def _tc_combine(text_emb, acc,
                W_g1, b_g1, W_g2, b_g2, W_fc1, b_fc1, W_fc2, b_fc2):
    nb = N // BLK
    row = lambda i: (i, 0)
    row_hi = lambda i: (nb + i, 0)
    full = lambda i: (0, 0)
    return pl.pallas_call(
        _tc_body,
        grid=(nb,),
        in_specs=[
            pl.BlockSpec((BLK, 256), row),
            pl.BlockSpec((BLK, DW), row),
            pl.BlockSpec((BLK, DW), row_hi),
            pl.BlockSpec((D1, D1), full),
            pl.BlockSpec((1, D1), full),
            pl.BlockSpec((D2, D2), full),
            pl.BlockSpec((1, D2), full),
            pl.BlockSpec((256, D1), full),
            pl.BlockSpec((1, D1), full),
            pl.BlockSpec((2 * D1 + D2, 256), full),
            pl.BlockSpec((1, 256), full),
        ],
        out_specs=pl.BlockSpec((BLK, 256), row),
        out_shape=jax.ShapeDtypeStruct((N, 256), jnp.float32),
    )(text_emb, acc, acc,
      W_g1, b_g1, W_g2, b_g2, W_fc1, b_fc1, W_fc2, b_fc2)


def kernel(text_emb, feature_2, graph_features, edge_index,
           W_g1, b_g1, W_g2, b_g2, W_fc1, b_fc1, W_fc2, b_fc2):
    ones = jnp.ones((N, 16), jnp.float32)
    x_lo = jnp.concatenate([graph_features[:, :DH], ones], axis=1)
    x_hi = jnp.concatenate([graph_features[:, DH:], feature_2, ones], axis=1)
    src2d = edge_index[0].reshape(E // CH, CH)
    dst2d = edge_index[1].reshape(E // CH, CH)
    acc = _sc_segment_sum(x_lo, x_hi, src2d, dst2d)
    return _tc_combine(
        text_emb, acc,
        W_g1, b_g1.reshape(1, -1), W_g2, b_g2.reshape(1, -1),
        W_fc1, b_fc1.reshape(1, -1), W_fc2, b_fc2.reshape(1, -1))


# speedup vs baseline: 12.6163x; 1.0034x over previous
import jax
import jax.numpy as jnp
from jax import lax
from jax.experimental import pallas as pl
from jax.experimental.pallas import tpu as pltpu
from jax.experimental.pallas import tpu_sc as plsc

N = 10000
E = 320000
D1 = 128
D2 = 64
DH = (D1 + D2) // 2
DW = DH + 16
NC = 2
NS = 16
CH = 80
EPT = E // NS
NCHUNK = EPT // CH
SEG = 50
ROWCHUNKS = N // CH
RR = (ROWCHUNKS + NS - 1) // NS
NBUF = 5


def _sc_body(xlo_hbm, xhi_hbm, src_hbm, dst_hbm, acc_hbm,
             acc_sp, srcseg_v, dstseg_v, r0_v, r1_v, r2_v, r3_v, r4_v,
             sem_g, sem_s):
    c = lax.axis_index("c")
    s = lax.axis_index("s")
    rows_bufs = (r0_v, r1_v, r2_v, r3_v, r4_v)

    zvec = jnp.zeros((16,), jnp.float32)

    def fill(i, carry):
        for k in range(DW // 16):
            r0_v[i, pl.ds(16 * k, 16)] = zvec
        return carry

    lax.fori_loop(0, CH, fill, 0)

    base_row = s * NCHUNK
    pltpu.sync_copy(src_hbm.at[pl.ds(base_row, SEG)], srcseg_v)
    pltpu.sync_copy(dst_hbm.at[pl.ds(base_row, SEG)], dstseg_v)

    def zinit(t, carry):
        chk = t * NS + s

        @pl.when(chk < ROWCHUNKS)
        def _():
            a0 = pl.multiple_of(chk * CH, 8)
            pltpu.async_copy(r0_v, acc_sp.at[pl.ds(a0, CH)], sem_g)

        return carry

    lax.fori_loop(0, RR, zinit, 0)

    def zwait(t, carry):
        chk = t * NS + s

        @pl.when(chk < ROWCHUNKS)
        def _():
            a0 = pl.multiple_of(chk * CH, 8)
            pltpu.make_async_copy(r0_v, acc_sp.at[pl.ds(a0, CH)], sem_g).wait()

        return carry

    lax.fori_loop(0, RR, zwait, 0)
    plsc.subcore_barrier()

    def start_gather(j, buf):
        @pl.when(c == 0)
        def _():
            pltpu.async_copy(xlo_hbm.at[srcseg_v.at[j % SEG]], buf, sem_g)

        @pl.when(c == 1)
        def _():
            pltpu.async_copy(xhi_hbm.at[srcseg_v.at[j % SEG]], buf, sem_g)

    def wait_gather(buf):
        pltpu.make_async_copy(xlo_hbm.at[srcseg_v.at[0]], buf, sem_g).wait()

    def start_scatter(j, buf):
        pltpu.async_copy(buf, acc_sp.at[dstseg_v.at[j % SEG]], sem_s, add=True)

    def wait_scatter(j, buf):
        pltpu.make_async_copy(buf, acc_sp.at[dstseg_v.at[j % SEG]], sem_s).wait()

    start_gather(0, rows_bufs[0])
    start_gather(1, rows_bufs[1])

    def outer(g, carry):
        for b in range(NBUF):
            j = g * NBUF + b

            if b == 0:
                reload_d = ((j % SEG) == 0) & (j > 0)

                @pl.when(reload_d)
                def _():
                    wait_scatter(j - 3, rows_bufs[(b - 3) % NBUF])
                    wait_scatter(j - 2, rows_bufs[(b - 2) % NBUF])
                    wait_scatter(j - 1, rows_bufs[(b - 1) % NBUF])
                    pltpu.sync_copy(dst_hbm.at[pl.ds(base_row + j, SEG)],
                                    dstseg_v)

                @pl.when(jnp.logical_not(reload_d) & (j >= 3))
                def _():
                    wait_scatter(j - 3, rows_bufs[(b - 3) % NBUF])
            elif b in (1, 2):
                @pl.when((j >= 3) & ((j % SEG) != b))
                def _():
                    wait_scatter(j - 3, rows_bufs[(b - 3) % NBUF])
            else:
                @pl.when(j >= 3)
                def _():
                    wait_scatter(j - 3, rows_bufs[(b - 3) % NBUF])

            if b == 3:
                reload_s = ((j % SEG) == SEG - 2) & (j + 2 < NCHUNK)

                @pl.when(reload_s)
                def _():
                    wait_gather(rows_bufs[b])
                    wait_gather(rows_bufs[(b + 1) % NBUF])
                    pltpu.sync_copy(src_hbm.at[pl.ds(base_row + j + 2, SEG)],
                                    srcseg_v)

            @pl.when(j + 2 < NCHUNK)
            def _():
                start_gather(j + 2, rows_bufs[(b + 2) % NBUF])

            if b == 3:
                @pl.when(jnp.logical_not(((j % SEG) == SEG - 2)
                                         & (j + 2 < NCHUNK)))
                def _():
                    wait_gather(rows_bufs[b])
            elif b == 4:
                @pl.when(jnp.logical_not(((j % SEG) == SEG - 1)
                                         & (j + 1 < NCHUNK)))
                def _():
                    wait_gather(rows_bufs[b])
            else:
                wait_gather(rows_bufs[b])

            start_scatter(j, rows_bufs[b])

        return carry

    lax.fori_loop(0, NCHUNK // NBUF, outer, 0)
    wait_scatter(NCHUNK - 3, rows_bufs[(NCHUNK - 3) % NBUF])
    wait_scatter(NCHUNK - 2, rows_bufs[(NCHUNK - 2) % NBUF])
    wait_scatter(NCHUNK - 1, rows_bufs[(NCHUNK - 1) % NBUF])
    plsc.subcore_barrier()

    def drain(t, carry):
        chk = t * NS + s

        @pl.when(chk < ROWCHUNKS)
        def _():
            a0 = pl.multiple_of(chk * CH, 8)
            o0 = pl.multiple_of(c * N + chk * CH, 8)
            pltpu.async_copy(acc_sp.at[pl.ds(a0, CH)],
                             acc_hbm.at[pl.ds(o0, CH)], sem_g)

        return carry

    lax.fori_loop(0, RR, drain, 0)

    def dwait(t, carry):
        chk = t * NS + s

        @pl.when(chk < ROWCHUNKS)
        def _():
            a0 = pl.multiple_of(chk * CH, 8)
            o0 = pl.multiple_of(c * N + chk * CH, 8)
            pltpu.make_async_copy(acc_sp.at[pl.ds(a0, CH)],
                                  acc_hbm.at[pl.ds(o0, CH)], sem_g).wait()

        return carry

    lax.fori_loop(0, RR, dwait, 0)


def _sc_segment_sum(x_lo, x_hi, src2d, dst2d):
    mesh = plsc.VectorSubcoreMesh(
        core_axis_name="c", subcore_axis_name="s", num_cores=NC, num_subcores=NS)
    f = pl.kernel(
        _sc_body,
        out_type=jax.ShapeDtypeStruct((NC * N, DW), jnp.float32),
        mesh=mesh,
        scratch_types=[
            pltpu.VMEM_SHARED((N, DW), jnp.float32),
            pltpu.VMEM((SEG, CH), jnp.int32),
            pltpu.VMEM((SEG, CH), jnp.int32),
            pltpu.VMEM((CH, DW), jnp.float32),
            pltpu.VMEM((CH, DW), jnp.float32),
            pltpu.VMEM((CH, DW), jnp.float32),
            pltpu.VMEM((CH, DW), jnp.float32),
            pltpu.VMEM((CH, DW), jnp.float32),
            pltpu.SemaphoreType.DMA,
            pltpu.SemaphoreType.DMA,
        ],
        compiler_params=pltpu.CompilerParams(use_tc_tiling_on_sc=False),
    )
    return f(x_lo, x_hi, src2d, dst2d)


BLK = 1000


def _tc_body(text_ref, acclo_ref, acchi_ref,
             wg1_ref, bg1_ref, wg2_ref, bg2_ref, wf1_ref, bf1_ref,
             wf2_ref, bf2_ref, out_ref):
    lo = acclo_ref[...]
    hi = acchi_ref[...]
    d = jnp.maximum(lo[:, DH:DH + 1], 1.0)
    g1 = jnp.concatenate([lo[:, :DH], hi[:, :D1 - DH]], axis=1) / d
    g2 = hi[:, D1 - DH:DH] / d
    h1 = jnp.maximum(
        jnp.dot(g1, wg1_ref[...], preferred_element_type=jnp.float32)
        + bg1_ref[...], 0.0)
    h2 = jnp.maximum(
        jnp.dot(g2, wg2_ref[...], preferred_element_type=jnp.float32)
        + bg2_ref[...], 0.0)
    t = (jnp.dot(text_ref[...], wf1_ref[...], preferred_element_type=jnp.float32)
         + bf1_ref[...])
    w2 = wf2_ref[...]
    out = jnp.dot(t, w2[:D1], preferred_element_type=jnp.float32)
    out = out + jnp.dot(h1, w2[D1:2 * D1], preferred_element_type=jnp.float32)
    out = out + jnp.dot(h2, w2[2 * D1:], preferred_element_type=jnp.float32)
    out_ref[...] = out + bf2_ref[...]


def _tc_combine(text_emb, acc,
                W_g1, b_g1, W_g2, b_g2, W_fc1, b_fc1, W_fc2, b_fc2):
    nb = N // BLK
    row = lambda i: (i, 0)
    row_hi = lambda i: (nb + i, 0)
    full = lambda i: (0, 0)
    return pl.pallas_call(
        _tc_body,
        grid=(nb,),
        in_specs=[
            pl.BlockSpec((BLK, 256), row),
            pl.BlockSpec((BLK, DW), row),
            pl.BlockSpec((BLK, DW), row_hi),
            pl.BlockSpec((D1, D1), full),
            pl.BlockSpec((1, D1), full),
            pl.BlockSpec((D2, D2), full),
            pl.BlockSpec((1, D2), full),
            pl.BlockSpec((256, D1), full),
            pl.BlockSpec((1, D1), full),
            pl.BlockSpec((2 * D1 + D2, 256), full),
            pl.BlockSpec((1, 256), full),
        ],
        out_specs=pl.BlockSpec((BLK, 256), row),
        out_shape=jax.ShapeDtypeStruct((N, 256), jnp.float32),
    )(text_emb, acc, acc,
      W_g1, b_g1, W_g2, b_g2, W_fc1, b_fc1, W_fc2, b_fc2)


def kernel(text_emb, feature_2, graph_features, edge_index,
           W_g1, b_g1, W_g2, b_g2, W_fc1, b_fc1, W_fc2, b_fc2):
    ones = jnp.ones((N, 16), jnp.float32)
    x_lo = jnp.concatenate([graph_features[:, :DH], ones], axis=1)
    x_hi = jnp.concatenate([graph_features[:, DH:], feature_2, ones], axis=1)
    src2d = edge_index[0].reshape(E // CH, CH)
    dst2d = edge_index[1].reshape(E // CH, CH)
    acc = _sc_segment_sum(x_lo, x_hi, src2d, dst2d)
    return _tc_combine(
        text_emb, acc,
        W_g1, b_g1.reshape(1, -1), W_g2, b_g2.reshape(1, -1),
        W_fc1, b_fc1.reshape(1, -1), W_fc2, b_fc2.reshape(1, -1))


# speedup vs baseline: 12.8296x; 1.0169x over previous
import jax
import jax.numpy as jnp
from jax import lax
from jax.experimental import pallas as pl
from jax.experimental.pallas import tpu as pltpu
from jax.experimental.pallas import tpu_sc as plsc

N = 10000
E = 320000
D1 = 128
D2 = 64
DH = (D1 + D2) // 2
DW = DH + 16
NC = 2
NS = 16
CH = 80
EPT = E // NS
NCHUNK = EPT // CH
SRCSEG = 125
DSEG = 25
ROWCHUNKS = N // CH
RR = (ROWCHUNKS + NS - 1) // NS
NBUF = 5


def _sc_body(xlo_hbm, xhi_hbm, src_hbm, dst_hbm, acc_hbm,
             acc_sp, srcseg_v, dsta_v, dstb_v, r0_v, r1_v, r2_v, r3_v, r4_v,
             sem_g, sem_s, sem_id):
    c = lax.axis_index("c")
    s = lax.axis_index("s")
    rows_bufs = (r0_v, r1_v, r2_v, r3_v, r4_v)

    zvec = jnp.zeros((16,), jnp.float32)

    def fill(i, carry):
        for k in range(DW // 16):
            r0_v[i, pl.ds(16 * k, 16)] = zvec
        return carry

    lax.fori_loop(0, CH, fill, 0)

    base_row = s * NCHUNK
    pltpu.sync_copy(src_hbm.at[pl.ds(base_row, SRCSEG)], srcseg_v)
    pltpu.sync_copy(dst_hbm.at[pl.ds(base_row, DSEG)], dsta_v)

    def zinit(t, carry):
        chk = t * NS + s

        @pl.when(chk < ROWCHUNKS)
        def _():
            a0 = pl.multiple_of(chk * CH, 8)
            pltpu.async_copy(r0_v, acc_sp.at[pl.ds(a0, CH)], sem_g)

        return carry

    lax.fori_loop(0, RR, zinit, 0)

    def zwait(t, carry):
        chk = t * NS + s

        @pl.when(chk < ROWCHUNKS)
        def _():
            a0 = pl.multiple_of(chk * CH, 8)
            pltpu.make_async_copy(r0_v, acc_sp.at[pl.ds(a0, CH)], sem_g).wait()

        return carry

    lax.fori_loop(0, RR, zwait, 0)
    plsc.subcore_barrier()

    def start_gather(j, buf):
        @pl.when(c == 0)
        def _():
            pltpu.async_copy(xlo_hbm.at[srcseg_v.at[j % SRCSEG]], buf, sem_g)

        @pl.when(c == 1)
        def _():
            pltpu.async_copy(xhi_hbm.at[srcseg_v.at[j % SRCSEG]], buf, sem_g)

    def wait_gather(buf):
        pltpu.make_async_copy(xlo_hbm.at[srcseg_v.at[0]], buf, sem_g).wait()

    def start_scatter(j, buf):
        p = (j // DSEG) % 2

        @pl.when(p == 0)
        def _():
            pltpu.async_copy(buf, acc_sp.at[dsta_v.at[j % DSEG]], sem_s,
                             add=True)

        @pl.when(p == 1)
        def _():
            pltpu.async_copy(buf, acc_sp.at[dstb_v.at[j % DSEG]], sem_s,
                             add=True)

    def wait_scatter(j, buf):
        p = (j // DSEG) % 2

        @pl.when(p == 0)
        def _():
            pltpu.make_async_copy(buf, acc_sp.at[dsta_v.at[j % DSEG]],
                                  sem_s).wait()

        @pl.when(p == 1)
        def _():
            pltpu.make_async_copy(buf, acc_sp.at[dstb_v.at[j % DSEG]],
                                  sem_s).wait()

    start_gather(0, rows_bufs[0])
    start_gather(1, rows_bufs[1])

    def outer(g, carry):
        for b in range(NBUF):
            j = g * NBUF + b

            if b == 0:
                p = (j // DSEG) % 2

                @pl.when((j % DSEG == 0) & (j > 0) & (p == 0))
                def _():
                    pltpu.make_async_copy(
                        dst_hbm.at[pl.ds(base_row + j, DSEG)], dsta_v,
                        sem_id).wait()

                @pl.when((j % DSEG == 0) & (j > 0) & (p == 1))
                def _():
                    pltpu.make_async_copy(
                        dst_hbm.at[pl.ds(base_row + j, DSEG)], dstb_v,
                        sem_id).wait()

            @pl.when(j >= 3)
            def _():
                wait_scatter(j - 3, rows_bufs[(b - 3) % NBUF])

            if b == 3:
                p = (j // DSEG) % 2

                @pl.when((j % DSEG == 3) & (j + 22 < NCHUNK) & (p == 0))
                def _():
                    pltpu.async_copy(
                        dst_hbm.at[pl.ds(base_row + j + 22, DSEG)], dstb_v,
                        sem_id)

                @pl.when((j % DSEG == 3) & (j + 22 < NCHUNK) & (p == 1))
                def _():
                    pltpu.async_copy(
                        dst_hbm.at[pl.ds(base_row + j + 22, DSEG)], dsta_v,
                        sem_id)

                @pl.when(j == SRCSEG - 2)
                def _():
                    wait_gather(rows_bufs[b])
                    wait_gather(rows_bufs[(b + 1) % NBUF])
                    pltpu.sync_copy(
                        src_hbm.at[pl.ds(base_row + SRCSEG, SRCSEG)],
                        srcseg_v)

            @pl.when(j + 2 < NCHUNK)
            def _():
                start_gather(j + 2, rows_bufs[(b + 2) % NBUF])

            if b == 3:
                @pl.when(j != SRCSEG - 2)
                def _():
                    wait_gather(rows_bufs[b])
            elif b == 4:
                @pl.when(j != SRCSEG - 1)
                def _():
                    wait_gather(rows_bufs[b])
            else:
                wait_gather(rows_bufs[b])

            start_scatter(j, rows_bufs[b])

        return carry

    lax.fori_loop(0, NCHUNK // NBUF, outer, 0)
    wait_scatter(NCHUNK - 3, rows_bufs[(NCHUNK - 3) % NBUF])
    wait_scatter(NCHUNK - 2, rows_bufs[(NCHUNK - 2) % NBUF])
    wait_scatter(NCHUNK - 1, rows_bufs[(NCHUNK - 1) % NBUF])
    plsc.subcore_barrier()

    def drain(t, carry):
        chk = t * NS + s

        @pl.when(chk < ROWCHUNKS)
        def _():
            a0 = pl.multiple_of(chk * CH, 8)
            o0 = pl.multiple_of(c * N + chk * CH, 8)
            pltpu.async_copy(acc_sp.at[pl.ds(a0, CH)],
                             acc_hbm.at[pl.ds(o0, CH)], sem_g)

        return carry

    lax.fori_loop(0, RR, drain, 0)

    def dwait(t, carry):
        chk = t * NS + s

        @pl.when(chk < ROWCHUNKS)
        def _():
            a0 = pl.multiple_of(chk * CH, 8)
            o0 = pl.multiple_of(c * N + chk * CH, 8)
            pltpu.make_async_copy(acc_sp.at[pl.ds(a0, CH)],
                                  acc_hbm.at[pl.ds(o0, CH)], sem_g).wait()

        return carry

    lax.fori_loop(0, RR, dwait, 0)


def _sc_segment_sum(x_lo, x_hi, src2d, dst2d):
    mesh = plsc.VectorSubcoreMesh(
        core_axis_name="c", subcore_axis_name="s", num_cores=NC, num_subcores=NS)
    f = pl.kernel(
        _sc_body,
        out_type=jax.ShapeDtypeStruct((NC * N, DW), jnp.float32),
        mesh=mesh,
        scratch_types=[
            pltpu.VMEM_SHARED((N, DW), jnp.float32),
            pltpu.VMEM((SRCSEG, CH), jnp.int32),
            pltpu.VMEM((DSEG, CH), jnp.int32),
            pltpu.VMEM((DSEG, CH), jnp.int32),
            pltpu.VMEM((CH, DW), jnp.float32),
            pltpu.VMEM((CH, DW), jnp.float32),
            pltpu.VMEM((CH, DW), jnp.float32),
            pltpu.VMEM((CH, DW), jnp.float32),
            pltpu.VMEM((CH, DW), jnp.float32),
            pltpu.SemaphoreType.DMA,
            pltpu.SemaphoreType.DMA,
            pltpu.SemaphoreType.DMA,
        ],
        compiler_params=pltpu.CompilerParams(use_tc_tiling_on_sc=False),
    )
    return f(x_lo, x_hi, src2d, dst2d)


BLK = 1000


def _tc_body(text_ref, acclo_ref, acchi_ref,
             wg1_ref, bg1_ref, wg2_ref, bg2_ref, wf1_ref, bf1_ref,
             wf2_ref, bf2_ref, out_ref):
    lo = acclo_ref[...]
    hi = acchi_ref[...]
    d = jnp.maximum(lo[:, DH:DH + 1], 1.0)
    g1 = jnp.concatenate([lo[:, :DH], hi[:, :D1 - DH]], axis=1) / d
    g2 = hi[:, D1 - DH:DH] / d
    h1 = jnp.maximum(
        jnp.dot(g1, wg1_ref[...], preferred_element_type=jnp.float32)
        + bg1_ref[...], 0.0)
    h2 = jnp.maximum(
        jnp.dot(g2, wg2_ref[...], preferred_element_type=jnp.float32)
        + bg2_ref[...], 0.0)
    t = (jnp.dot(text_ref[...], wf1_ref[...], preferred_element_type=jnp.float32)
         + bf1_ref[...])
    w2 = wf2_ref[...]
    out = jnp.dot(t, w2[:D1], preferred_element_type=jnp.float32)
    out = out + jnp.dot(h1, w2[D1:2 * D1], preferred_element_type=jnp.float32)
    out = out + jnp.dot(h2, w2[2 * D1:], preferred_element_type=jnp.float32)
    out_ref[...] = out + bf2_ref[...]


def _tc_combine(text_emb, acc,
                W_g1, b_g1, W_g2, b_g2, W_fc1, b_fc1, W_fc2, b_fc2):
    nb = N // BLK
    row = lambda i: (i, 0)
    row_hi = lambda i: (nb + i, 0)
    full = lambda i: (0, 0)
    return pl.pallas_call(
        _tc_body,
        grid=(nb,),
        in_specs=[
            pl.BlockSpec((BLK, 256), row),
            pl.BlockSpec((BLK, DW), row),
            pl.BlockSpec((BLK, DW), row_hi),
            pl.BlockSpec((D1, D1), full),
            pl.BlockSpec((1, D1), full),
            pl.BlockSpec((D2, D2), full),
            pl.BlockSpec((1, D2), full),
            pl.BlockSpec((256, D1), full),
            pl.BlockSpec((1, D1), full),
            pl.BlockSpec((2 * D1 + D2, 256), full),
            pl.BlockSpec((1, 256), full),
        ],
        out_specs=pl.BlockSpec((BLK, 256), row),
        out_shape=jax.ShapeDtypeStruct((N, 256), jnp.float32),
    )(text_emb, acc, acc,
      W_g1, b_g1, W_g2, b_g2, W_fc1, b_fc1, W_fc2, b_fc2)


def kernel(text_emb, feature_2, graph_features, edge_index,
           W_g1, b_g1, W_g2, b_g2, W_fc1, b_fc1, W_fc2, b_fc2):
    ones = jnp.ones((N, 16), jnp.float32)
    x_lo = jnp.concatenate([graph_features[:, :DH], ones], axis=1)
    x_hi = jnp.concatenate([graph_features[:, DH:], feature_2, ones], axis=1)
    src2d = edge_index[0].reshape(E // CH, CH)
    dst2d = edge_index[1].reshape(E // CH, CH)
    acc = _sc_segment_sum(x_lo, x_hi, src2d, dst2d)
    return _tc_combine(
        text_emb, acc,
        W_g1, b_g1.reshape(1, -1), W_g2, b_g2.reshape(1, -1),
        W_fc1, b_fc1.reshape(1, -1), W_fc2, b_fc2.reshape(1, -1))


# speedup vs baseline: 12.8680x; 1.0030x over previous
import jax
import jax.numpy as jnp
from jax import lax
from jax.experimental import pallas as pl
from jax.experimental.pallas import tpu as pltpu
from jax.experimental.pallas import tpu_sc as plsc

N = 10000
E = 320000
D1 = 128
D2 = 64
DH = (D1 + D2) // 2
DW = DH + 16
NC = 2
NS = 16
CH = 80
EPT = E // NS
NCHUNK = EPT // CH
SRCSEG = 125
DSEG = 25
ROWCHUNKS = N // CH
RR = (ROWCHUNKS + NS - 1) // NS
NBUF = 5


def _sc_body(xlo_hbm, xhi_hbm, src_hbm, dst_hbm, acc_hbm,
             acc_sp, srcseg_v, dsta_v, dstb_v, r0_v, r1_v, r2_v, r3_v, r4_v,
             sem_g, sem_s, sem_id):
    c = lax.axis_index("c")
    s = lax.axis_index("s")
    rows_bufs = (r0_v, r1_v, r2_v, r3_v, r4_v)

    zvec = jnp.zeros((16,), jnp.float32)

    def fill(i, carry):
        for k in range(DW // 16):
            r0_v[i, pl.ds(16 * k, 16)] = zvec
        return carry

    lax.fori_loop(0, CH, fill, 0)

    base_row = s * NCHUNK
    pltpu.sync_copy(src_hbm.at[pl.ds(base_row, SRCSEG)], srcseg_v)
    pltpu.sync_copy(dst_hbm.at[pl.ds(base_row, DSEG)], dsta_v)

    def zinit(t, carry):
        chk = t * NS + s

        @pl.when(chk < ROWCHUNKS)
        def _():
            a0 = pl.multiple_of(chk * CH, 8)
            pltpu.async_copy(r0_v, acc_sp.at[pl.ds(a0, CH)], sem_g)

        return carry

    lax.fori_loop(0, RR, zinit, 0)

    def zwait(t, carry):
        chk = t * NS + s

        @pl.when(chk < ROWCHUNKS)
        def _():
            a0 = pl.multiple_of(chk * CH, 8)
            pltpu.make_async_copy(r0_v, acc_sp.at[pl.ds(a0, CH)], sem_g).wait()

        return carry

    lax.fori_loop(0, RR, zwait, 0)
    plsc.subcore_barrier()

    def start_gather(j, buf):
        @pl.when(c == 0)
        def _():
            pltpu.async_copy(xlo_hbm.at[srcseg_v.at[j % SRCSEG]], buf, sem_g)

        @pl.when(c == 1)
        def _():
            pltpu.async_copy(xhi_hbm.at[srcseg_v.at[j % SRCSEG]], buf, sem_g)

    def wait_gather(buf):
        pltpu.make_async_copy(xlo_hbm.at[srcseg_v.at[0]], buf, sem_g).wait()

    def start_scatter(j, buf):
        p = (j // DSEG) % 2

        @pl.when(p == 0)
        def _():
            pltpu.async_copy(buf, acc_sp.at[dsta_v.at[j % DSEG]], sem_s,
                             add=True)

        @pl.when(p == 1)
        def _():
            pltpu.async_copy(buf, acc_sp.at[dstb_v.at[j % DSEG]], sem_s,
                             add=True)

    def wait_scatter(j, buf):
        p = (j // DSEG) % 2

        @pl.when(p == 0)
        def _():
            pltpu.make_async_copy(buf, acc_sp.at[dsta_v.at[j % DSEG]],
                                  sem_s).wait()

        @pl.when(p == 1)
        def _():
            pltpu.make_async_copy(buf, acc_sp.at[dstb_v.at[j % DSEG]],
                                  sem_s).wait()

    start_gather(0, rows_bufs[0])
    start_gather(1, rows_bufs[1])
    start_gather(2, rows_bufs[2])

    def outer(g, carry):
        for b in range(NBUF):
            j = g * NBUF + b

            if b == 0:
                p = (j // DSEG) % 2

                @pl.when((j % DSEG == 0) & (j > 0) & (p == 0))
                def _():
                    pltpu.make_async_copy(
                        dst_hbm.at[pl.ds(base_row + j, DSEG)], dsta_v,
                        sem_id).wait()

                @pl.when((j % DSEG == 0) & (j > 0) & (p == 1))
                def _():
                    pltpu.make_async_copy(
                        dst_hbm.at[pl.ds(base_row + j, DSEG)], dstb_v,
                        sem_id).wait()

            @pl.when(j >= 2)
            def _():
                wait_scatter(j - 2, rows_bufs[(b - 2) % NBUF])

            if b == 2:
                p = (j // DSEG) % 2

                @pl.when((j % DSEG == 2) & (j + 23 < NCHUNK) & (p == 0))
                def _():
                    pltpu.async_copy(
                        dst_hbm.at[pl.ds(base_row + j + 23, DSEG)], dstb_v,
                        sem_id)

                @pl.when((j % DSEG == 2) & (j + 23 < NCHUNK) & (p == 1))
                def _():
                    pltpu.async_copy(
                        dst_hbm.at[pl.ds(base_row + j + 23, DSEG)], dsta_v,
                        sem_id)

                @pl.when(j == SRCSEG - 3)
                def _():
                    wait_gather(rows_bufs[b])
                    wait_gather(rows_bufs[(b + 1) % NBUF])
                    wait_gather(rows_bufs[(b + 2) % NBUF])
                    pltpu.sync_copy(
                        src_hbm.at[pl.ds(base_row + SRCSEG, SRCSEG)],
                        srcseg_v)

            @pl.when(j + 3 < NCHUNK)
            def _():
                start_gather(j + 3, rows_bufs[(b + 3) % NBUF])

            if b == 2:
                @pl.when(j != SRCSEG - 3)
                def _():
                    wait_gather(rows_bufs[b])
            elif b == 3:
                @pl.when(j != SRCSEG - 2)
                def _():
                    wait_gather(rows_bufs[b])
            elif b == 4:
                @pl.when(j != SRCSEG - 1)
                def _():
                    wait_gather(rows_bufs[b])
            else:
                wait_gather(rows_bufs[b])

            start_scatter(j, rows_bufs[b])

        return carry

    lax.fori_loop(0, NCHUNK // NBUF, outer, 0)
    wait_scatter(NCHUNK - 2, rows_bufs[(NCHUNK - 2) % NBUF])
    wait_scatter(NCHUNK - 1, rows_bufs[(NCHUNK - 1) % NBUF])
    plsc.subcore_barrier()

    def drain(t, carry):
        chk = t * NS + s

        @pl.when(chk < ROWCHUNKS)
        def _():
            a0 = pl.multiple_of(chk * CH, 8)
            o0 = pl.multiple_of(c * N + chk * CH, 8)
            pltpu.async_copy(acc_sp.at[pl.ds(a0, CH)],
                             acc_hbm.at[pl.ds(o0, CH)], sem_g)

        return carry

    lax.fori_loop(0, RR, drain, 0)

    def dwait(t, carry):
        chk = t * NS + s

        @pl.when(chk < ROWCHUNKS)
        def _():
            a0 = pl.multiple_of(chk * CH, 8)
            o0 = pl.multiple_of(c * N + chk * CH, 8)
            pltpu.make_async_copy(acc_sp.at[pl.ds(a0, CH)],
                                  acc_hbm.at[pl.ds(o0, CH)], sem_g).wait()

        return carry

    lax.fori_loop(0, RR, dwait, 0)


def _sc_segment_sum(x_lo, x_hi, src2d, dst2d):
    mesh = plsc.VectorSubcoreMesh(
        core_axis_name="c", subcore_axis_name="s", num_cores=NC, num_subcores=NS)
    f = pl.kernel(
        _sc_body,
        out_type=jax.ShapeDtypeStruct((NC * N, DW), jnp.float32),
        mesh=mesh,
        scratch_types=[
            pltpu.VMEM_SHARED((N, DW), jnp.float32),
            pltpu.VMEM((SRCSEG, CH), jnp.int32),
            pltpu.VMEM((DSEG, CH), jnp.int32),
            pltpu.VMEM((DSEG, CH), jnp.int32),
            pltpu.VMEM((CH, DW), jnp.float32),
            pltpu.VMEM((CH, DW), jnp.float32),
            pltpu.VMEM((CH, DW), jnp.float32),
            pltpu.VMEM((CH, DW), jnp.float32),
            pltpu.VMEM((CH, DW), jnp.float32),
            pltpu.SemaphoreType.DMA,
            pltpu.SemaphoreType.DMA,
            pltpu.SemaphoreType.DMA,
        ],
        compiler_params=pltpu.CompilerParams(use_tc_tiling_on_sc=False),
    )
    return f(x_lo, x_hi, src2d, dst2d)


BLK = 1000


def _tc_body(text_ref, acclo_ref, acchi_ref,
             wg1_ref, bg1_ref, wg2_ref, bg2_ref, wf1_ref, bf1_ref,
             wf2_ref, bf2_ref, out_ref):
    lo = acclo_ref[...]
    hi = acchi_ref[...]
    d = jnp.maximum(lo[:, DH:DH + 1], 1.0)
    g1 = jnp.concatenate([lo[:, :DH], hi[:, :D1 - DH]], axis=1) / d
    g2 = hi[:, D1 - DH:DH] / d
    h1 = jnp.maximum(
        jnp.dot(g1, wg1_ref[...], preferred_element_type=jnp.float32)
        + bg1_ref[...], 0.0)
    h2 = jnp.maximum(
        jnp.dot(g2, wg2_ref[...], preferred_element_type=jnp.float32)
        + bg2_ref[...], 0.0)
    t = (jnp.dot(text_ref[...], wf1_ref[...], preferred_element_type=jnp.float32)
         + bf1_ref[...])
    w2 = wf2_ref[...]
    out = jnp.dot(t, w2[:D1], preferred_element_type=jnp.float32)
    out = out + jnp.dot(h1, w2[D1:2 * D1], preferred_element_type=jnp.float32)
    out = out + jnp.dot(h2, w2[2 * D1:], preferred_element_type=jnp.float32)
    out_ref[...] = out + bf2_ref[...]


def _tc_combine(text_emb, acc,
                W_g1, b_g1, W_g2, b_g2, W_fc1, b_fc1, W_fc2, b_fc2):
    nb = N // BLK
    row = lambda i: (i, 0)
    row_hi = lambda i: (nb + i, 0)
    full = lambda i: (0, 0)
    return pl.pallas_call(
        _tc_body,
        grid=(nb,),
        in_specs=[
            pl.BlockSpec((BLK, 256), row),
            pl.BlockSpec((BLK, DW), row),
            pl.BlockSpec((BLK, DW), row_hi),
            pl.BlockSpec((D1, D1), full),
            pl.BlockSpec((1, D1), full),
            pl.BlockSpec((D2, D2), full),
            pl.BlockSpec((1, D2), full),
            pl.BlockSpec((256, D1), full),
            pl.BlockSpec((1, D1), full),
            pl.BlockSpec((2 * D1 + D2, 256), full),
            pl.BlockSpec((1, 256), full),
        ],
        out_specs=pl.BlockSpec((BLK, 256), row),
        out_shape=jax.ShapeDtypeStruct((N, 256), jnp.float32),
    )(text_emb, acc, acc,
      W_g1, b_g1, W_g2, b_g2, W_fc1, b_fc1, W_fc2, b_fc2)


def kernel(text_emb, feature_2, graph_features, edge_index,
           W_g1, b_g1, W_g2, b_g2, W_fc1, b_fc1, W_fc2, b_fc2):
    ones = jnp.ones((N, 16), jnp.float32)
    x_lo = jnp.concatenate([graph_features[:, :DH], ones], axis=1)
    x_hi = jnp.concatenate([graph_features[:, DH:], feature_2, ones], axis=1)
    src2d = edge_index[0].reshape(E // CH, CH)
    dst2d = edge_index[1].reshape(E // CH, CH)
    acc = _sc_segment_sum(x_lo, x_hi, src2d, dst2d)
    return _tc_combine(
        text_emb, acc,
        W_g1, b_g1.reshape(1, -1), W_g2, b_g2.reshape(1, -1),
        W_fc1, b_fc1.reshape(1, -1), W_fc2, b_fc2.reshape(1, -1))


# speedup vs baseline: 13.0557x; 1.0146x over previous
import jax
import jax.numpy as jnp
from jax import lax
from jax.experimental import pallas as pl
from jax.experimental.pallas import tpu as pltpu
from jax.experimental.pallas import tpu_sc as plsc

N = 10000
E = 320000
D1 = 128
D2 = 64
DH = (D1 + D2) // 2
DW = DH + 16
NC = 2
NS = 16
CH = 80
EPT = E // NS
NCHUNK = EPT // CH
SRCSEG = 125
DSEG = 25
ROWCHUNKS = N // CH
RR = (ROWCHUNKS + NS - 1) // NS
NBUF = 5


def _sc_body(xlo_hbm, xhi_hbm, src_hbm, dst_hbm, acc_hbm,
             acc_sp, srcseg_v, dsta_v, dstb_v, r0_v, r1_v, r2_v, r3_v, r4_v,
             sem_g, sem_s, sem_id):
    c = lax.axis_index("c")
    s = lax.axis_index("s")
    rows_bufs = (r0_v, r1_v, r2_v, r3_v, r4_v)

    zvec = jnp.zeros((16,), jnp.float32)

    def fill(i, carry):
        for k in range(DW // 16):
            r0_v[i, pl.ds(16 * k, 16)] = zvec
        return carry

    lax.fori_loop(0, CH, fill, 0)

    base_row = s * NCHUNK
    pltpu.sync_copy(src_hbm.at[pl.ds(base_row, SRCSEG)], srcseg_v)
    pltpu.sync_copy(dst_hbm.at[pl.ds(base_row, DSEG)], dsta_v)

    def zinit(t, carry):
        chk = t * NS + s

        @pl.when(chk < ROWCHUNKS)
        def _():
            a0 = pl.multiple_of(chk * CH, 8)
            pltpu.async_copy(r0_v, acc_sp.at[pl.ds(a0, CH)], sem_g)

        return carry

    lax.fori_loop(0, RR, zinit, 0)

    def zwait(t, carry):
        chk = t * NS + s

        @pl.when(chk < ROWCHUNKS)
        def _():
            a0 = pl.multiple_of(chk * CH, 8)
            pltpu.make_async_copy(r0_v, acc_sp.at[pl.ds(a0, CH)], sem_g).wait()

        return carry

    lax.fori_loop(0, RR, zwait, 0)
    plsc.subcore_barrier()

    def start_gather(j, buf):
        @pl.when(c == 0)
        def _():
            pltpu.async_copy(xlo_hbm.at[srcseg_v.at[j % SRCSEG]], buf, sem_g)

        @pl.when(c == 1)
        def _():
            pltpu.async_copy(xhi_hbm.at[srcseg_v.at[j % SRCSEG]], buf, sem_g)

    def wait_gather(buf):
        pltpu.make_async_copy(xlo_hbm.at[srcseg_v.at[0]], buf, sem_g).wait()

    def start_scatter(j, buf):
        p = (j // DSEG) % 2

        @pl.when(p == 0)
        def _():
            pltpu.async_copy(buf, acc_sp.at[dsta_v.at[j % DSEG]], sem_s,
                             add=True)

        @pl.when(p == 1)
        def _():
            pltpu.async_copy(buf, acc_sp.at[dstb_v.at[j % DSEG]], sem_s,
                             add=True)

    def wait_scatter(j, buf):
        p = (j // DSEG) % 2

        @pl.when(p == 0)
        def _():
            pltpu.make_async_copy(buf, acc_sp.at[dsta_v.at[j % DSEG]],
                                  sem_s).wait()

        @pl.when(p == 1)
        def _():
            pltpu.make_async_copy(buf, acc_sp.at[dstb_v.at[j % DSEG]],
                                  sem_s).wait()

    start_gather(0, rows_bufs[0])
    start_gather(1, rows_bufs[1])
    start_gather(2, rows_bufs[2])

    def outer(g, carry):
        for b in range(NBUF):
            j = g * NBUF + b

            if b == 0:
                p = (j // DSEG) % 2

                @pl.when((j % DSEG == 0) & (j > 0) & (p == 0))
                def _():
                    pltpu.make_async_copy(
                        dst_hbm.at[pl.ds(base_row + j, DSEG)], dsta_v,
                        sem_id).wait()

                @pl.when((j % DSEG == 0) & (j > 0) & (p == 1))
                def _():
                    pltpu.make_async_copy(
                        dst_hbm.at[pl.ds(base_row + j, DSEG)], dstb_v,
                        sem_id).wait()

            @pl.when(j >= 2)
            def _():
                wait_scatter(j - 2, rows_bufs[(b - 2) % NBUF])

            if b == 2:
                p = (j // DSEG) % 2

                @pl.when((j % DSEG == 2) & (j + 23 < NCHUNK) & (p == 0))
                def _():
                    pltpu.async_copy(
                        dst_hbm.at[pl.ds(base_row + j + 23, DSEG)], dstb_v,
                        sem_id)

                @pl.when((j % DSEG == 2) & (j + 23 < NCHUNK) & (p == 1))
                def _():
                    pltpu.async_copy(
                        dst_hbm.at[pl.ds(base_row + j + 23, DSEG)], dsta_v,
                        sem_id)

                @pl.when(j == SRCSEG - 3)
                def _():
                    wait_gather(rows_bufs[b])
                    wait_gather(rows_bufs[(b + 1) % NBUF])
                    wait_gather(rows_bufs[(b + 2) % NBUF])
                    pltpu.sync_copy(
                        src_hbm.at[pl.ds(base_row + SRCSEG, SRCSEG)],
                        srcseg_v)

            @pl.when(j + 3 < NCHUNK)
            def _():
                start_gather(j + 3, rows_bufs[(b + 3) % NBUF])

            if b == 2:
                @pl.when(j != SRCSEG - 3)
                def _():
                    wait_gather(rows_bufs[b])
            elif b == 3:
                @pl.when(j != SRCSEG - 2)
                def _():
                    wait_gather(rows_bufs[b])
            elif b == 4:
                @pl.when(j != SRCSEG - 1)
                def _():
                    wait_gather(rows_bufs[b])
            else:
                wait_gather(rows_bufs[b])

            start_scatter(j, rows_bufs[b])

        return carry

    lax.fori_loop(0, NCHUNK // NBUF, outer, 0)
    wait_scatter(NCHUNK - 2, rows_bufs[(NCHUNK - 2) % NBUF])
    wait_scatter(NCHUNK - 1, rows_bufs[(NCHUNK - 1) % NBUF])
    plsc.subcore_barrier()

    def drain(t, carry):
        chk = t * NS + s

        @pl.when(chk < ROWCHUNKS)
        def _():
            a0 = pl.multiple_of(chk * CH, 8)
            o0 = pl.multiple_of(c * N + chk * CH, 8)
            pltpu.async_copy(acc_sp.at[pl.ds(a0, CH)],
                             acc_hbm.at[pl.ds(o0, CH)], sem_g)

        return carry

    lax.fori_loop(0, RR, drain, 0)

    def dwait(t, carry):
        chk = t * NS + s

        @pl.when(chk < ROWCHUNKS)
        def _():
            a0 = pl.multiple_of(chk * CH, 8)
            o0 = pl.multiple_of(c * N + chk * CH, 8)
            pltpu.make_async_copy(acc_sp.at[pl.ds(a0, CH)],
                                  acc_hbm.at[pl.ds(o0, CH)], sem_g).wait()

        return carry

    lax.fori_loop(0, RR, dwait, 0)


def _sc_segment_sum(x_lo, x_hi, src2d, dst2d):
    mesh = plsc.VectorSubcoreMesh(
        core_axis_name="c", subcore_axis_name="s", num_cores=NC, num_subcores=NS)
    f = pl.kernel(
        _sc_body,
        out_type=jax.ShapeDtypeStruct((NC * N, DW), jnp.float32),
        mesh=mesh,
        scratch_types=[
            pltpu.VMEM_SHARED((N, DW), jnp.float32),
            pltpu.VMEM((SRCSEG, CH), jnp.int32),
            pltpu.VMEM((DSEG, CH), jnp.int32),
            pltpu.VMEM((DSEG, CH), jnp.int32),
            pltpu.VMEM((CH, DW), jnp.float32),
            pltpu.VMEM((CH, DW), jnp.float32),
            pltpu.VMEM((CH, DW), jnp.float32),
            pltpu.VMEM((CH, DW), jnp.float32),
            pltpu.VMEM((CH, DW), jnp.float32),
            pltpu.SemaphoreType.DMA,
            pltpu.SemaphoreType.DMA,
            pltpu.SemaphoreType.DMA,
        ],
        compiler_params=pltpu.CompilerParams(use_tc_tiling_on_sc=False),
    )
    return f(x_lo, x_hi, src2d, dst2d)


BLK = 2000


def _tc_body(text_ref, acclo_ref, acchi_ref,
             wg1_ref, bg1_ref, wg2_ref, bg2_ref, wf1_ref, bf1_ref,
             wf2_ref, bf2_ref, out_ref):
    lo = acclo_ref[...]
    hi = acchi_ref[...]
    d = jnp.maximum(lo[:, DH:DH + 1], 1.0)
    g1 = jnp.concatenate([lo[:, :DH], hi[:, :D1 - DH]], axis=1) / d
    g2 = hi[:, D1 - DH:DH] / d
    h1 = jnp.maximum(
        jnp.dot(g1, wg1_ref[...], preferred_element_type=jnp.float32)
        + bg1_ref[...], 0.0)
    h2 = jnp.maximum(
        jnp.dot(g2, wg2_ref[...], preferred_element_type=jnp.float32)
        + bg2_ref[...], 0.0)
    t = (jnp.dot(text_ref[...], wf1_ref[...], preferred_element_type=jnp.float32)
         + bf1_ref[...])
    w2 = wf2_ref[...]
    out = jnp.dot(t, w2[:D1], preferred_element_type=jnp.float32)
    out = out + jnp.dot(h1, w2[D1:2 * D1], preferred_element_type=jnp.float32)
    out = out + jnp.dot(h2, w2[2 * D1:], preferred_element_type=jnp.float32)
    out_ref[...] = out + bf2_ref[...]


def _tc_combine(text_emb, acc,
                W_g1, b_g1, W_g2, b_g2, W_fc1, b_fc1, W_fc2, b_fc2):
    nb = N // BLK
    row = lambda i: (i, 0)
    row_hi = lambda i: (nb + i, 0)
    full = lambda i: (0, 0)
    return pl.pallas_call(
        _tc_body,
        grid=(nb,),
        in_specs=[
            pl.BlockSpec((BLK, 256), row),
            pl.BlockSpec((BLK, DW), row),
            pl.BlockSpec((BLK, DW), row_hi),
            pl.BlockSpec((D1, D1), full),
            pl.BlockSpec((1, D1), full),
            pl.BlockSpec((D2, D2), full),
            pl.BlockSpec((1, D2), full),
            pl.BlockSpec((256, D1), full),
            pl.BlockSpec((1, D1), full),
            pl.BlockSpec((2 * D1 + D2, 256), full),
            pl.BlockSpec((1, 256), full),
        ],
        out_specs=pl.BlockSpec((BLK, 256), row),
        out_shape=jax.ShapeDtypeStruct((N, 256), jnp.float32),
    )(text_emb, acc, acc,
      W_g1, b_g1, W_g2, b_g2, W_fc1, b_fc1, W_fc2, b_fc2)


def kernel(text_emb, feature_2, graph_features, edge_index,
           W_g1, b_g1, W_g2, b_g2, W_fc1, b_fc1, W_fc2, b_fc2):
    ones = jnp.ones((N, 16), jnp.float32)
    x_lo = jnp.concatenate([graph_features[:, :DH], ones], axis=1)
    x_hi = jnp.concatenate([graph_features[:, DH:], feature_2, ones], axis=1)
    src2d = edge_index[0].reshape(E // CH, CH)
    dst2d = edge_index[1].reshape(E // CH, CH)
    acc = _sc_segment_sum(x_lo, x_hi, src2d, dst2d)
    return _tc_combine(
        text_emb, acc,
        W_g1, b_g1.reshape(1, -1), W_g2, b_g2.reshape(1, -1),
        W_fc1, b_fc1.reshape(1, -1), W_fc2, b_fc2.reshape(1, -1))
